# baseline (speedup 1.0000x reference)
import functools
import jax
import jax.numpy as jnp
from jax import lax
from jax.experimental import pallas as pl
from jax.experimental.pallas import tpu as pltpu, tpu_sc as plsc

_B = 16384
_F = 26
_V = 100000
_D = 64

_NW = 32
_BF = _B * _F
_PER_W = _BF // _NW
_CHUNK = 128
_NCH = _PER_W // _CHUNK
_NVEC = _PER_W // 16


def _make_kernel():
    mesh = plsc.VectorSubcoreMesh(core_axis_name="c", subcore_axis_name="s")

    @functools.partial(
        pl.kernel,
        mesh=mesh,
        out_type=jax.ShapeDtypeStruct((_NW * _NCH, _CHUNK, _D), jnp.float32),
        scratch_types=[
            pltpu.VMEM((_NCH, _CHUNK), jnp.int32),
            pltpu.VMEM((_CHUNK, _D), jnp.float32),
            pltpu.SemaphoreType.DMA,
        ],
        compiler_params=pltpu.CompilerParams(use_tc_tiling_on_sc=False),
    )
    def k(idx_hbm, table_hbm, out_hbm, idx_v, rows_v, sem):
        wid = lax.axis_index("s") * 2 + lax.axis_index("c")
        pltpu.sync_copy(idx_hbm.at[wid], idx_v)

        lane = lax.iota(jnp.int32, 16)

        def fix(j, carry):
            r = j // (_CHUNK // 16)
            col = (j % (_CHUNK // 16)) * 16
            pos16 = lax.rem(j * 16, _F)
            f = lax.rem(pos16 + lane, _F)
            v = idx_v[r, pl.ds(col, 16)]
            idx_v[r, pl.ds(col, 16)] = f * _V + v
            return carry

        lax.fori_loop(0, _NVEC, fix, 0)

        def chunk(r, carry):
            pltpu.async_copy(table_hbm.at[idx_v.at[r]], rows_v, sem).wait()
            pltpu.sync_copy(rows_v, out_hbm.at[wid * _NCH + r])
            return carry

        lax.fori_loop(0, _NCH, chunk, 0)

    return k


_kernel_call = _make_kernel()


@jax.jit
def kernel(indices, tables):
    idx3 = indices.reshape(_NW, _NCH, _CHUNK)
    table_flat = tables.reshape(_F * _V, _D)
    out = _kernel_call(idx3, table_flat)
    return out.reshape(_B, _F, _D)


# speedup vs baseline: 1.0430x; 1.0430x over previous
import functools
import jax
import jax.numpy as jnp
from jax import lax
from jax.experimental import pallas as pl
from jax.experimental.pallas import tpu as pltpu, tpu_sc as plsc

_B = 16384
_F = 26
_V = 100000
_D = 64

_NW = 32
_BF = _B * _F
_PER_W = _BF // _NW
_CHUNK = 128
_NCH = _PER_W // _CHUNK
_SCH = 256
_CPS = _SCH // _CHUNK
_NSUP = _PER_W // _SCH
_NBUF = 4


def _make_kernel():
    mesh = plsc.VectorSubcoreMesh(core_axis_name="c", subcore_axis_name="s")

    @functools.partial(
        pl.kernel,
        mesh=mesh,
        out_type=jax.ShapeDtypeStruct((_NW * _NSUP, _SCH, _D), jnp.float32),
        scratch_types=[
            pltpu.VMEM((_NCH, _CHUNK), jnp.int32),
            pltpu.VMEM((_NBUF, _SCH, _D), jnp.float32),
            [pltpu.SemaphoreType.DMA] * _NBUF,
            [pltpu.SemaphoreType.DMA] * _NBUF,
        ],
        compiler_params=pltpu.CompilerParams(use_tc_tiling_on_sc=False),
    )
    def k(idx_hbm, table_hbm, out_hbm, idx_v, rows_v, gsems, wsems):
        wid = lax.axis_index("s") * 2 + lax.axis_index("c")
        pltpu.sync_copy(idx_hbm.at[wid], idx_v)
        lane = lax.iota(jnp.int32, 16)
        obase = wid * _NSUP

        def fire(S, slot):
            for c in range(_CPS):
                r = S * _CPS + c
                for u in range(_CHUNK // 16):
                    col = u * 16
                    f = lax.rem((r * (_CHUNK // 16) + u) * 16 + lane, _F)
                    idx_v[r, pl.ds(col, 16)] = f * _V + idx_v[r, pl.ds(col, 16)]
                pltpu.async_copy(
                    table_hbm.at[idx_v.at[r]],
                    rows_v.at[slot, pl.ds(c * _CHUNK, _CHUNK)],
                    gsems[slot])

        def drain(slot, sem):
            pltpu.make_async_copy(out_hbm.at[0], rows_v.at[slot], sem).wait()

        fire(0, 0)
        fire(1, 1)

        def step(S, slot):
            drain(slot, gsems[slot])
            pltpu.async_copy(rows_v.at[slot], out_hbm.at[obase + S],
                             wsems[slot])
            nslot = (slot + 2) % _NBUF

            @pl.when(S >= 2)
            def _():
                drain(nslot, wsems[nslot])

            @pl.when(S + 2 < _NSUP)
            def _():
                fire(S + 2, nslot)

        def outer(t, carry):
            for b in range(_NBUF):
                step(t * _NBUF + b, b)
            return carry

        lax.fori_loop(0, _NSUP // _NBUF, outer, 0)
        s2 = (_NSUP - 2) % _NBUF
        s1 = (_NSUP - 1) % _NBUF
        drain(s2, wsems[s2])
        drain(s1, wsems[s1])

    return k


_kernel_call = _make_kernel()


@jax.jit
def kernel(indices, tables):
    idx3 = indices.reshape(_NW, _NCH, _CHUNK)
    table_flat = tables.reshape(_F * _V, _D)
    out = _kernel_call(idx3, table_flat)
    return out.reshape(_B, _F, _D)
